# Initial kernel scaffold; baseline (speedup 1.0000x reference)
import jax
import jax.numpy as jnp
from jax.experimental import pallas as pl


def kernel(alphax_idx, alphay_idx, theta_idx, phi_idx, LUT):
    raise NotImplementedError("write your pallas kernel here")


# speedup vs baseline: 18.8915x; 18.8915x over previous
import functools

import jax
import jax.numpy as jnp
from jax import lax
from jax.experimental import pallas as pl
from jax.experimental.pallas import tpu as pltpu
from jax.experimental.pallas import tpu_sc as plsc

N = 262144
V = 16 * 16 * 16 * 16
DPAD = 16
LANES = 16

NUM_CORES = 2
NUM_SUBCORES = 16
NW = NUM_CORES * NUM_SUBCORES
B_W = N // NW
CHUNK = 2048
NCHUNK = B_W // CHUNK


def _gather_body(ax_hbm, ay_hbm, th_hbm, ph_hbm, lut_hbm, out_hbm,
                 ax_v, ay_v, th_v, ph_v, lin_v, rows_v, sem):
    wid = lax.axis_index("s") * NUM_CORES + lax.axis_index("c")
    base = wid * B_W

    for c in range(NCHUNK):
        off = base + c * CHUNK
        pltpu.sync_copy(ax_hbm.at[pl.ds(off, CHUNK)], ax_v)
        pltpu.sync_copy(ay_hbm.at[pl.ds(off, CHUNK)], ay_v)
        pltpu.sync_copy(th_hbm.at[pl.ds(off, CHUNK)], th_v)
        pltpu.sync_copy(ph_hbm.at[pl.ds(off, CHUNK)], ph_v)

        def body(i, _):
            s = pl.ds(i * LANES, LANES)
            lin = (
                (ax_v[s] << 12) | (ay_v[s] << 8) | (th_v[s] << 4) | ph_v[s]
            )
            lin_v[s] = lin
            return _

        lax.fori_loop(0, CHUNK // LANES, body, None)

        pltpu.async_copy(lut_hbm.at[lin_v], rows_v, sem).wait()
        pltpu.sync_copy(rows_v, out_hbm.at[pl.ds(off, CHUNK)])


@functools.partial(jax.jit, static_argnums=())
def kernel(alphax_idx, alphay_idx, theta_idx, phi_idx, LUT):
    ax = alphax_idx.astype(jnp.int32)
    ay = alphay_idx.astype(jnp.int32)
    th = theta_idx.astype(jnp.int32)
    ph = phi_idx.astype(jnp.int32)

    lut_rows = LUT.reshape(V, 9)
    lut_pad = jnp.pad(lut_rows, ((0, 0), (0, DPAD - 9)))

    mesh = plsc.VectorSubcoreMesh(core_axis_name="c", subcore_axis_name="s")
    out = pl.kernel(
        _gather_body,
        mesh=mesh,
        compiler_params=pltpu.CompilerParams(use_tc_tiling_on_sc=False),
        out_type=jax.ShapeDtypeStruct((N, DPAD), jnp.float32),
        scratch_types=[
            pltpu.VMEM((CHUNK,), jnp.int32),
            pltpu.VMEM((CHUNK,), jnp.int32),
            pltpu.VMEM((CHUNK,), jnp.int32),
            pltpu.VMEM((CHUNK,), jnp.int32),
            pltpu.VMEM((CHUNK,), jnp.int32),
            pltpu.VMEM((CHUNK, DPAD), jnp.float32),
            pltpu.SemaphoreType.DMA,
        ],
    )(ax, ay, th, ph, lut_pad)

    return out[:, :9].reshape(N, 3, 3)
